# Initial kernel scaffold; baseline (speedup 1.0000x reference)
import jax
import jax.numpy as jnp
from jax.experimental import pallas as pl


def kernel(x, w_attn, w_proj):
    raise NotImplementedError("write your pallas kernel here")


# speedup vs baseline: 8.2393x; 8.2393x over previous
import functools
import math

import jax
import jax.numpy as jnp
from jax import lax
from jax.experimental import pallas as pl
from jax.experimental.pallas import tpu as pltpu

_MASK_VALUE = -1e30
_VMEM_LIMIT = 48 * 1024 * 1024


def _attn_pair_kernel(x_ref, w_ref, y_ref, *, scale, tq, d):
    x = x_ref[...].astype(jnp.bfloat16)
    qkv = jnp.dot(x, w_ref[...], preferred_element_type=jnp.float32)
    T = x.shape[0]
    nq = T // tq

    outs = []
    for j in range(2):
        qh = (qkv[:, j * d:(j + 1) * d] * scale).astype(jnp.bfloat16)
        kh = qkv[:, (2 + j) * d:(3 + j) * d].astype(jnp.bfloat16)
        vh = qkv[:, (4 + j) * d:(5 + j) * d].astype(jnp.bfloat16)
        ys = []
        for qi in range(nq):
            lo = qi * tq
            qt = qh[lo:lo + tq]
            sd = lax.dot_general(qt, kh[lo:lo + tq],
                                 (((1,), (1,)), ((), ())),
                                 preferred_element_type=jnp.float32)
            row = lax.broadcasted_iota(jnp.int32, (tq, tq), 0)
            col = lax.broadcasted_iota(jnp.int32, (tq, tq), 1)
            sd = jnp.where(col <= row, sd, _MASK_VALUE)
            if qi == 0:
                m = jnp.max(sd, axis=-1, keepdims=True)
                pd = jnp.exp(sd - m)
                l = jnp.sum(pd, axis=-1, keepdims=True)
                acc = lax.dot_general(pd.astype(jnp.bfloat16), vh[lo:lo + tq],
                                      (((1,), (0,)), ((), ())),
                                      preferred_element_type=jnp.float32)
            else:
                sp = lax.dot_general(qt, kh[:lo],
                                     (((1,), (1,)), ((), ())),
                                     preferred_element_type=jnp.float32)
                m = jnp.maximum(jnp.max(sp, axis=-1, keepdims=True),
                                jnp.max(sd, axis=-1, keepdims=True))
                pp = jnp.exp(sp - m)
                pd = jnp.exp(sd - m)
                l = (jnp.sum(pp, axis=-1, keepdims=True) +
                     jnp.sum(pd, axis=-1, keepdims=True))
                acc = lax.dot_general(pp.astype(jnp.bfloat16), vh[:lo],
                                      (((1,), (0,)), ((), ())),
                                      preferred_element_type=jnp.float32)
                acc = acc + lax.dot_general(pd.astype(jnp.bfloat16),
                                            vh[lo:lo + tq],
                                            (((1,), (0,)), ((), ())),
                                            preferred_element_type=jnp.float32)
            y = acc * pl.reciprocal(l, approx=True)
            ys.append(y.astype(jnp.bfloat16))
        outs.append(jnp.concatenate(ys, axis=0))
    y_ref[...] = jnp.concatenate(outs, axis=1)


def _proj_kernel(y_ref, w_ref, o_ref):
    o_ref[...] = jnp.dot(y_ref[...], w_ref[...],
                         preferred_element_type=jnp.float32)


def kernel(x, w_attn, w_proj):
    B, T, C = x.shape
    n_head = 12
    D = C // n_head
    HP = n_head // 2
    scale = 1.0 / math.sqrt(D)
    tq = 256 if T % 256 == 0 else T

    w_pair = (w_attn.astype(jnp.bfloat16).T
              .reshape(C, 3, HP, 2 * D)
              .transpose(0, 2, 1, 3)
              .reshape(C, HP * 6 * D))
    w_out = w_proj.astype(jnp.bfloat16).T

    attn_cost = pl.CostEstimate(
        flops=2 * B * T * C * 3 * C + 2 * 2 * B * n_head * T * T * D,
        transcendentals=B * n_head * T * T * 5 // 8,
        bytes_accessed=(B * T * C * 4) + (3 * C * C * 2) + (B * T * C * 2),
    )
    y = pl.pallas_call(
        functools.partial(_attn_pair_kernel, scale=scale, tq=tq, d=D),
        out_shape=jax.ShapeDtypeStruct((B, T, C), jnp.bfloat16),
        grid=(B, HP),
        in_specs=[
            pl.BlockSpec((None, T, C), lambda b, hp: (b, 0, 0)),
            pl.BlockSpec((C, 6 * D), lambda b, hp: (0, hp)),
        ],
        out_specs=pl.BlockSpec((None, T, 2 * D), lambda b, hp: (b, 0, hp)),
        compiler_params=pltpu.CompilerParams(
            dimension_semantics=("parallel", "parallel"),
            vmem_limit_bytes=_VMEM_LIMIT,
        ),
        cost_estimate=attn_cost,
    )(x, w_pair)

    M = B * T
    tm = 512 if M % 512 == 0 else M
    proj_cost = pl.CostEstimate(
        flops=2 * M * C * C,
        transcendentals=0,
        bytes_accessed=(M * C * 2) + (C * C * 2) + (M * C * 4),
    )
    out = pl.pallas_call(
        _proj_kernel,
        out_shape=jax.ShapeDtypeStruct((M, C), jnp.float32),
        grid=(M // tm,),
        in_specs=[
            pl.BlockSpec((tm, C), lambda i: (i, 0)),
            pl.BlockSpec((C, C), lambda i: (0, 0)),
        ],
        out_specs=pl.BlockSpec((tm, C), lambda i: (i, 0)),
        compiler_params=pltpu.CompilerParams(
            dimension_semantics=("parallel",),
            vmem_limit_bytes=_VMEM_LIMIT,
        ),
        cost_estimate=proj_cost,
    )(y.reshape(M, C), w_out)
    return out.reshape(B, T, C)


# speedup vs baseline: 10.2363x; 1.2424x over previous
import functools
import math

import jax
import jax.numpy as jnp
from jax import lax
from jax.experimental import pallas as pl
from jax.experimental.pallas import tpu as pltpu

_MASK_VALUE = -1e30
_VMEM_LIMIT = 48 * 1024 * 1024


def _attn_pair_kernel(x_ref, w_ref, y_ref, xb_ref, *, tq, d):
    hp = pl.program_id(1)

    @pl.when(hp == 0)
    def _cast_x():
        xb_ref[...] = x_ref[...].astype(jnp.bfloat16)

    qkv = jnp.dot(xb_ref[...], w_ref[...],
                  preferred_element_type=jnp.float32)
    T = qkv.shape[0]
    nq = T // tq
    row = lax.broadcasted_iota(jnp.int32, (tq, tq), 0)
    col = lax.broadcasted_iota(jnp.int32, (tq, tq), 1)
    adder = jnp.where(col <= row, 0.0, _MASK_VALUE)
    ones_col = jnp.ones((T, 1), jnp.bfloat16)

    qp = qkv[:, 0:2 * d]
    kp = qkv[:, 2 * d:4 * d]
    qb = qp.astype(jnp.bfloat16)
    kb = kp.astype(jnp.bfloat16)
    vb = qkv[:, 4 * d:6 * d].astype(jnp.bfloat16)
    seg = lax.broadcasted_iota(jnp.int32, (2 * d, 2), 0) // d
    hid = lax.broadcasted_iota(jnp.int32, (2 * d, 2), 1)
    blockdiag = jnp.where(seg == hid, 1.0, 0.0)
    qn2 = lax.dot_general(qp * qp, blockdiag, (((1,), (0,)), ((), ())),
                          preferred_element_type=jnp.float32)
    kn2 = lax.dot_general(kp * kp, blockdiag, (((1,), (0,)), ((), ())),
                          preferred_element_type=jnp.float32)
    k2max = jnp.max(kn2, axis=0, keepdims=True)
    mpair = jnp.sqrt(qn2 * k2max)

    outs = []
    for j in range(2):
        qh = qb[:, j * d:(j + 1) * d]
        kh = kb[:, j * d:(j + 1) * d]
        ve = jnp.concatenate([vb[:, j * d:(j + 1) * d], ones_col],
                             axis=1)
        mrow = mpair[:, j:j + 1]
        ys = []
        for qi in range(nq):
            lo = qi * tq
            qt = qh[lo:lo + tq]
            mq = mrow[lo:lo + tq]
            sd = lax.dot_general(qt, kh[lo:lo + tq],
                                 (((1,), (1,)), ((), ())),
                                 preferred_element_type=jnp.float32)
            pd = jnp.exp((sd - mq) + adder).astype(jnp.bfloat16)
            ya = lax.dot_general(pd, ve[lo:lo + tq],
                                 (((1,), (0,)), ((), ())),
                                 preferred_element_type=jnp.float32)
            if qi > 0:
                sp = lax.dot_general(qt, kh[:lo], (((1,), (1,)), ((), ())),
                                     preferred_element_type=jnp.float32)
                pp = jnp.exp(sp - mq).astype(jnp.bfloat16)
                ya = ya + lax.dot_general(pp, ve[:lo],
                                          (((1,), (0,)), ((), ())),
                                          preferred_element_type=jnp.float32)
            y = ya[:, :d] * pl.reciprocal(ya[:, d:d + 1], approx=True)
            ys.append(y.astype(jnp.bfloat16))
        outs.append(jnp.concatenate(ys, axis=0))
    y_ref[...] = jnp.concatenate(outs, axis=1)


def _proj_kernel(y_ref, w_ref, o_ref):
    o_ref[...] = jnp.dot(y_ref[...], w_ref[...],
                         preferred_element_type=jnp.float32)


def kernel(x, w_attn, w_proj):
    B, T, C = x.shape
    n_head = 12
    D = C // n_head
    HP = n_head // 2
    scale = 1.0 / math.sqrt(D)
    tq = 256 if T % 256 == 0 else T

    w_scaled = jnp.concatenate([w_attn[:C] * scale, w_attn[C:]], axis=0)
    w_packed = (w_scaled.astype(jnp.bfloat16).T
                .reshape(C, 3, HP, 2 * D)
                .transpose(0, 2, 1, 3)
                .reshape(C, 3 * C))
    w_out = w_proj.astype(jnp.bfloat16).T

    attn_cost = pl.CostEstimate(
        flops=2 * B * T * C * 3 * C + 2 * 2 * B * n_head * T * T * D,
        transcendentals=B * n_head * T * T * 5 // 8,
        bytes_accessed=(B * T * C * 4) + (3 * C * C * 2) + (B * T * C * 2),
    )
    y = pl.pallas_call(
        functools.partial(_attn_pair_kernel, tq=tq, d=D),
        out_shape=jax.ShapeDtypeStruct((B, T, C), jnp.bfloat16),
        grid=(B, HP),
        in_specs=[
            pl.BlockSpec((None, T, C), lambda b, hp: (b, 0, 0)),
            pl.BlockSpec((C, 6 * D), lambda b, hp: (0, hp)),
        ],
        out_specs=pl.BlockSpec((None, T, 2 * D), lambda b, hp: (b, 0, hp)),
        scratch_shapes=[pltpu.VMEM((T, C), jnp.bfloat16)],
        compiler_params=pltpu.CompilerParams(
            dimension_semantics=("parallel", "arbitrary"),
            vmem_limit_bytes=_VMEM_LIMIT,
        ),
        cost_estimate=attn_cost,
    )(x, w_packed)

    M = B * T
    tm = 512 if M % 512 == 0 else M
    proj_cost = pl.CostEstimate(
        flops=2 * M * C * C,
        transcendentals=0,
        bytes_accessed=(M * C * 2) + (C * C * 2) + (M * C * 4),
    )
    out = pl.pallas_call(
        _proj_kernel,
        out_shape=jax.ShapeDtypeStruct((M, C), jnp.float32),
        grid=(M // tm,),
        in_specs=[
            pl.BlockSpec((tm, C), lambda i: (i, 0)),
            pl.BlockSpec((C, C), lambda i: (0, 0)),
        ],
        out_specs=pl.BlockSpec((tm, C), lambda i: (i, 0)),
        compiler_params=pltpu.CompilerParams(
            dimension_semantics=("parallel",),
            vmem_limit_bytes=_VMEM_LIMIT,
        ),
        cost_estimate=proj_cost,
    )(y.reshape(M, C), w_out)
    return out.reshape(B, T, C)


# speedup vs baseline: 10.7785x; 1.0530x over previous
import functools
import math

import jax
import jax.numpy as jnp
from jax import lax
from jax.experimental import pallas as pl
from jax.experimental.pallas import tpu as pltpu

_MASK_VALUE = -1e30
_VMEM_LIMIT = 48 * 1024 * 1024


def _attn_pair_kernel(x_ref, w_ref, wo_ref, o_ref, xb_ref, y_ref, *, tq, d):
    hp = pl.program_id(1)

    @pl.when(hp == 0)
    def _cast_x():
        xb_ref[...] = x_ref[...].astype(jnp.bfloat16)

    qkv = jnp.dot(xb_ref[...], w_ref[...],
                  preferred_element_type=jnp.float32)
    T = qkv.shape[0]
    nq = T // tq
    row = lax.broadcasted_iota(jnp.int32, (tq, tq), 0)
    col = lax.broadcasted_iota(jnp.int32, (tq, tq), 1)
    adder = jnp.where(col <= row, 0.0, _MASK_VALUE)
    ones_col = jnp.ones((T, 1), jnp.bfloat16)

    qp = qkv[:, 0:2 * d]
    kp = qkv[:, 2 * d:4 * d]
    qb = qp.astype(jnp.bfloat16)
    kb = kp.astype(jnp.bfloat16)
    vb = qkv[:, 4 * d:6 * d].astype(jnp.bfloat16)
    seg = lax.broadcasted_iota(jnp.int32, (2 * d, 2), 0) // d
    hid = lax.broadcasted_iota(jnp.int32, (2 * d, 2), 1)
    blockdiag = jnp.where(seg == hid, 1.0, 0.0)
    qn2 = lax.dot_general(qp * qp, blockdiag, (((1,), (0,)), ((), ())),
                          preferred_element_type=jnp.float32)
    kn2 = lax.dot_general(kp * kp, blockdiag, (((1,), (0,)), ((), ())),
                          preferred_element_type=jnp.float32)
    k2max = jnp.max(kn2, axis=0, keepdims=True)
    mpair = jnp.sqrt(qn2 * k2max)

    outs = []
    for j in range(2):
        qh = qb[:, j * d:(j + 1) * d]
        kh = kb[:, j * d:(j + 1) * d]
        ve = jnp.concatenate([vb[:, j * d:(j + 1) * d], ones_col],
                             axis=1)
        mrow = mpair[:, j:j + 1]
        ys = []
        for qi in range(nq):
            lo = qi * tq
            qt = qh[lo:lo + tq]
            mq = mrow[lo:lo + tq]
            sd = lax.dot_general(qt, kh[lo:lo + tq],
                                 (((1,), (1,)), ((), ())),
                                 preferred_element_type=jnp.float32)
            pd = jnp.exp((sd - mq) + adder).astype(jnp.bfloat16)
            ya = lax.dot_general(pd, ve[lo:lo + tq],
                                 (((1,), (0,)), ((), ())),
                                 preferred_element_type=jnp.float32)
            if qi > 0:
                sp = lax.dot_general(qt, kh[:lo], (((1,), (1,)), ((), ())),
                                     preferred_element_type=jnp.float32)
                pp = jnp.exp(sp - mq).astype(jnp.bfloat16)
                ya = ya + lax.dot_general(pp, ve[:lo],
                                          (((1,), (0,)), ((), ())),
                                          preferred_element_type=jnp.float32)
            y = ya[:, :d] * pl.reciprocal(ya[:, d:d + 1], approx=True)
            ys.append(y.astype(jnp.bfloat16))
        outs.append(jnp.concatenate(ys, axis=0))
    y_ref[:, pl.ds(hp * 2 * d, 2 * d)] = jnp.concatenate(outs, axis=1)

    @pl.when(hp == pl.num_programs(1) - 1)
    def _project():
        o_ref[...] = jnp.dot(y_ref[...], wo_ref[...],
                             preferred_element_type=jnp.float32)


def kernel(x, w_attn, w_proj):
    B, T, C = x.shape
    n_head = 12
    D = C // n_head
    HP = n_head // 2
    scale = 1.0 / math.sqrt(D)
    tq = 256 if T % 256 == 0 else T

    w_scaled = jnp.concatenate([w_attn[:C] * scale, w_attn[C:]], axis=0)
    w_packed = (w_scaled.astype(jnp.bfloat16).T
                .reshape(C, 3, HP, 2 * D)
                .transpose(0, 2, 1, 3)
                .reshape(C, 3 * C))
    w_out = w_proj.astype(jnp.bfloat16).T

    attn_cost = pl.CostEstimate(
        flops=(2 * B * T * C * 3 * C + 2 * 2 * B * n_head * T * T * D
               + 2 * B * T * C * C),
        transcendentals=B * n_head * T * T * 5 // 8,
        bytes_accessed=(B * T * C * 4) + (4 * C * C * 2) + (B * T * C * 4),
    )
    out = pl.pallas_call(
        functools.partial(_attn_pair_kernel, tq=tq, d=D),
        out_shape=jax.ShapeDtypeStruct((B, T, C), jnp.float32),
        grid=(B, HP),
        in_specs=[
            pl.BlockSpec((None, T, C), lambda b, hp: (b, 0, 0)),
            pl.BlockSpec((C, 6 * D), lambda b, hp: (0, hp)),
            pl.BlockSpec((C, C), lambda b, hp: (0, 0)),
        ],
        out_specs=pl.BlockSpec((None, T, C), lambda b, hp: (b, 0, 0)),
        scratch_shapes=[
            pltpu.VMEM((T, C), jnp.bfloat16),
            pltpu.VMEM((T, C), jnp.bfloat16),
        ],
        compiler_params=pltpu.CompilerParams(
            dimension_semantics=("parallel", "arbitrary"),
            vmem_limit_bytes=_VMEM_LIMIT,
        ),
        cost_estimate=attn_cost,
    )(x, w_packed, w_out)
    return out


# speedup vs baseline: 11.0423x; 1.0245x over previous
import functools
import math

import jax
import jax.numpy as jnp
from jax import lax
from jax.experimental import pallas as pl
from jax.experimental.pallas import tpu as pltpu

_MASK_VALUE = -1e30
_VMEM_LIMIT = 48 * 1024 * 1024


def _attn_pair_kernel(x_ref, w_ref, wo_ref, o_ref, xb_ref, y_ref, *, tq, d):
    hp = pl.program_id(1)

    @pl.when(hp == 0)
    def _cast_x():
        xb_ref[...] = x_ref[...].astype(jnp.bfloat16)

    qkv = jnp.dot(xb_ref[...], w_ref[...],
                  preferred_element_type=jnp.float32
                  ).astype(jnp.bfloat16)
    T = qkv.shape[0]
    nq = T // tq
    row = lax.broadcasted_iota(jnp.int32, (tq, tq), 0)
    col = lax.broadcasted_iota(jnp.int32, (tq, tq), 1)
    adder = jnp.where(col <= row, 0.0, _MASK_VALUE)
    ones_col = jnp.ones((T, 1), jnp.bfloat16)

    qb = qkv[:, 0:2 * d]
    kb = qkv[:, 2 * d:4 * d]
    vb = qkv[:, 4 * d:6 * d]
    seg = lax.broadcasted_iota(jnp.int32, (2 * d, 2), 0) // d
    hid = lax.broadcasted_iota(jnp.int32, (2 * d, 2), 1)
    blockdiag = jnp.where(seg == hid, 1.0, 0.0).astype(jnp.bfloat16)
    qn2 = lax.dot_general(qb * qb, blockdiag, (((1,), (0,)), ((), ())),
                          preferred_element_type=jnp.float32)
    kn2 = lax.dot_general(kb * kb, blockdiag, (((1,), (0,)), ((), ())),
                          preferred_element_type=jnp.float32)
    k2max = jnp.max(kn2, axis=0, keepdims=True)
    mpair = jnp.sqrt(qn2 * k2max * 1.01)

    ves = [jnp.concatenate([vb[:, j * d:(j + 1) * d], ones_col], axis=1)
           for j in range(2)]
    ys = [[], []]
    for qi in range(nq):
        lo = qi * tq
        for j in range(2):
            qh = qb[:, j * d:(j + 1) * d]
            kh = kb[:, j * d:(j + 1) * d]
            ve = ves[j]
            qt = qh[lo:lo + tq]
            mq = mpair[lo:lo + tq, j:j + 1]
            sd = lax.dot_general(qt, kh[lo:lo + tq],
                                 (((1,), (1,)), ((), ())),
                                 preferred_element_type=jnp.float32)
            pd = jnp.exp((sd - mq) + adder).astype(jnp.bfloat16)
            ya = lax.dot_general(pd, ve[lo:lo + tq],
                                 (((1,), (0,)), ((), ())),
                                 preferred_element_type=jnp.float32)
            if qi > 0:
                sp = lax.dot_general(qt, kh[:lo], (((1,), (1,)), ((), ())),
                                     preferred_element_type=jnp.float32)
                pp = jnp.exp(sp - mq).astype(jnp.bfloat16)
                ya = ya + lax.dot_general(pp, ve[:lo],
                                          (((1,), (0,)), ((), ())),
                                          preferred_element_type=jnp.float32)
            y = ya[:, :d] * pl.reciprocal(ya[:, d:d + 1], approx=True)
            ys[j].append(y.astype(jnp.bfloat16))
    outs = [jnp.concatenate(t, axis=0) for t in ys]
    y_ref[:, pl.ds(hp * 2 * d, 2 * d)] = jnp.concatenate(outs, axis=1)

    @pl.when(hp == pl.num_programs(1) - 1)
    def _project():
        o_ref[...] = jnp.dot(y_ref[...], wo_ref[...],
                             preferred_element_type=jnp.float32)


def kernel(x, w_attn, w_proj):
    B, T, C = x.shape
    n_head = 12
    D = C // n_head
    HP = n_head // 2
    scale = 1.0 / math.sqrt(D)
    tq = 256 if T % 256 == 0 else T

    scale_vec = jnp.concatenate(
        [jnp.full((C, 1), scale, jnp.float32),
         jnp.ones((2 * C, 1), jnp.float32)])
    w_packed = ((w_attn * scale_vec).astype(jnp.bfloat16).T
                .reshape(C, 3, HP, 2 * D)
                .transpose(0, 2, 1, 3)
                .reshape(C, 3 * C))
    w_out = w_proj.astype(jnp.bfloat16).T

    attn_cost = pl.CostEstimate(
        flops=(2 * B * T * C * 3 * C + 2 * 2 * B * n_head * T * T * D
               + 2 * B * T * C * C),
        transcendentals=B * n_head * T * T * 5 // 8,
        bytes_accessed=(B * T * C * 4) + (4 * C * C * 2) + (B * T * C * 4),
    )
    out = pl.pallas_call(
        functools.partial(_attn_pair_kernel, tq=tq, d=D),
        out_shape=jax.ShapeDtypeStruct((B, T, C), jnp.float32),
        grid=(B, HP),
        in_specs=[
            pl.BlockSpec((None, T, C), lambda b, hp: (b, 0, 0)),
            pl.BlockSpec((C, 6 * D), lambda b, hp: (0, hp)),
            pl.BlockSpec((C, C), lambda b, hp: (0, 0)),
        ],
        out_specs=pl.BlockSpec((None, T, C), lambda b, hp: (b, 0, 0)),
        scratch_shapes=[
            pltpu.VMEM((T, C), jnp.bfloat16),
            pltpu.VMEM((T, C), jnp.bfloat16),
        ],
        compiler_params=pltpu.CompilerParams(
            dimension_semantics=("parallel", "arbitrary"),
            vmem_limit_bytes=_VMEM_LIMIT,
        ),
        cost_estimate=attn_cost,
    )(x, w_packed, w_out)
    return out


# speedup vs baseline: 12.4321x; 1.1259x over previous
import functools
import math

import jax
import jax.numpy as jnp
from jax import lax
from jax.experimental import pallas as pl
from jax.experimental.pallas import tpu as pltpu

_MASK_VALUE = -1e30
_VMEM_LIMIT = 48 * 1024 * 1024


def _attn_pair_kernel(x_ref, w_ref, wo_ref, o_ref, xb_ref, y_ref, *, tq, d):
    hp = pl.program_id(1)

    @pl.when(hp == 0)
    def _cast_x():
        xb_ref[...] = x_ref[...].astype(jnp.bfloat16)

    qkv = jnp.dot(xb_ref[...], w_ref[...],
                  preferred_element_type=jnp.float32
                  ).astype(jnp.bfloat16)
    T = qkv.shape[0]
    nq = T // tq
    row = lax.broadcasted_iota(jnp.int32, (tq, tq), 0)
    col = lax.broadcasted_iota(jnp.int32, (tq, tq), 1)
    adder = jnp.where(col <= row, 0.0, _MASK_VALUE)
    ones_col = jnp.ones((T, 1), jnp.bfloat16)

    qb = qkv[:, 0:2 * d]
    kb = qkv[:, 2 * d:4 * d]
    vb = qkv[:, 4 * d:6 * d]
    seg = lax.broadcasted_iota(jnp.int32, (2 * d, 2), 0) // d
    hid = lax.broadcasted_iota(jnp.int32, (2 * d, 2), 1)
    blockdiag = jnp.where(seg == hid, 1.0, 0.0).astype(jnp.bfloat16)
    qn2 = lax.dot_general(qb * qb, blockdiag, (((1,), (0,)), ((), ())),
                          preferred_element_type=jnp.float32)
    kn2 = lax.dot_general(kb * kb, blockdiag, (((1,), (0,)), ((), ())),
                          preferred_element_type=jnp.float32)
    k2max = jnp.max(kn2, axis=0, keepdims=True)
    mpair = jnp.sqrt(qn2 * k2max * 1.01)

    ves = [jnp.concatenate([vb[:, j * d:(j + 1) * d], ones_col], axis=1)
           for j in range(2)]
    units = [(qi, j) for qi in range(nq) for j in range(2)]
    sds, sps, mqs = {}, {}, {}
    for qi, j in units:
        lo = qi * tq
        qt = qb[lo:lo + tq, j * d:(j + 1) * d]
        kh = kb[:, j * d:(j + 1) * d]
        mqs[qi, j] = mpair[lo:lo + tq, j:j + 1]
        sds[qi, j] = lax.dot_general(qt, kh[lo:lo + tq],
                                     (((1,), (1,)), ((), ())),
                                     preferred_element_type=jnp.float32)
        if qi > 0:
            sps[qi, j] = lax.dot_general(qt, kh[:lo], (((1,), (1,)), ((), ())),
                                         preferred_element_type=jnp.float32)
    pds, pps = {}, {}
    for qi, j in units:
        pds[qi, j] = jnp.exp((sds[qi, j] - mqs[qi, j]) + adder
                             ).astype(jnp.bfloat16)
        if qi > 0:
            pps[qi, j] = jnp.exp(sps[qi, j] - mqs[qi, j]).astype(jnp.bfloat16)
    ys = [[], []]
    for qi, j in units:
        lo = qi * tq
        ya = lax.dot_general(pds[qi, j], ves[j][lo:lo + tq],
                             (((1,), (0,)), ((), ())),
                             preferred_element_type=jnp.float32)
        if qi > 0:
            ya = ya + lax.dot_general(pps[qi, j], ves[j][:lo],
                                      (((1,), (0,)), ((), ())),
                                      preferred_element_type=jnp.float32)
        y = ya[:, :d] * pl.reciprocal(ya[:, d:d + 1], approx=True)
        ys[j].append(y.astype(jnp.bfloat16))
    outs = [jnp.concatenate(t, axis=0) for t in ys]
    y_ref[:, pl.ds(hp * 2 * d, 2 * d)] = jnp.concatenate(outs, axis=1)

    @pl.when(hp == pl.num_programs(1) - 1)
    def _project():
        o_ref[...] = jnp.dot(y_ref[...], wo_ref[...],
                             preferred_element_type=jnp.float32)


def kernel(x, w_attn, w_proj):
    B, T, C = x.shape
    n_head = 12
    D = C // n_head
    HP = n_head // 2
    scale = 1.0 / math.sqrt(D)
    tq = 256 if T % 256 == 0 else T

    scale_vec = jnp.concatenate(
        [jnp.full((C, 1), scale, jnp.float32),
         jnp.ones((2 * C, 1), jnp.float32)])
    w_packed = ((w_attn * scale_vec).astype(jnp.bfloat16).T
                .reshape(C, 3, HP, 2 * D)
                .transpose(0, 2, 1, 3)
                .reshape(C, 3 * C))
    w_out = w_proj.astype(jnp.bfloat16).T

    attn_cost = pl.CostEstimate(
        flops=(2 * B * T * C * 3 * C + 2 * 2 * B * n_head * T * T * D
               + 2 * B * T * C * C),
        transcendentals=B * n_head * T * T * 5 // 8,
        bytes_accessed=(B * T * C * 4) + (4 * C * C * 2) + (B * T * C * 4),
    )
    out = pl.pallas_call(
        functools.partial(_attn_pair_kernel, tq=tq, d=D),
        out_shape=jax.ShapeDtypeStruct((B, T, C), jnp.float32),
        grid=(B, HP),
        in_specs=[
            pl.BlockSpec((None, T, C), lambda b, hp: (b, 0, 0)),
            pl.BlockSpec((C, 6 * D), lambda b, hp: (0, hp)),
            pl.BlockSpec((C, C), lambda b, hp: (0, 0)),
        ],
        out_specs=pl.BlockSpec((None, T, C), lambda b, hp: (b, 0, 0)),
        scratch_shapes=[
            pltpu.VMEM((T, C), jnp.bfloat16),
            pltpu.VMEM((T, C), jnp.bfloat16),
        ],
        compiler_params=pltpu.CompilerParams(
            dimension_semantics=("parallel", "arbitrary"),
            vmem_limit_bytes=_VMEM_LIMIT,
        ),
        cost_estimate=attn_cost,
    )(x, w_packed, w_out)
    return out


# speedup vs baseline: 12.6914x; 1.0209x over previous
import functools
import math

import jax
import jax.numpy as jnp
from jax import lax
from jax.experimental import pallas as pl
from jax.experimental.pallas import tpu as pltpu

_VMEM_LIMIT = 48 * 1024 * 1024


def _attn_pair_kernel(x_ref, w_ref, wo_ref, o_ref, xb_ref, y_ref, *, tq, d):
    hp = pl.program_id(1)

    @pl.when(hp == 0)
    def _cast_x():
        xb_ref[...] = x_ref[...].astype(jnp.bfloat16)

    qkv = jnp.dot(xb_ref[...], w_ref[...],
                  preferred_element_type=jnp.float32
                  ).astype(jnp.bfloat16)
    T = qkv.shape[0]
    nq = T // tq
    row = lax.broadcasted_iota(jnp.int32, (tq, tq), 0)
    col = lax.broadcasted_iota(jnp.int32, (tq, tq), 1)
    maskb = jnp.where(col <= row, 1.0, 0.0).astype(jnp.bfloat16)
    ones_col = jnp.ones((T, 1), jnp.bfloat16)

    qb = qkv[:, 0:2 * d]
    kb = qkv[:, 2 * d:4 * d]
    vb = qkv[:, 4 * d:6 * d]
    seg = lax.broadcasted_iota(jnp.int32, (2 * d, 2), 0) // d
    hid = lax.broadcasted_iota(jnp.int32, (2 * d, 2), 1)
    blockdiag = jnp.where(seg == hid, 1.0, 0.0).astype(jnp.bfloat16)
    qn2 = lax.dot_general(qb * qb, blockdiag, (((1,), (0,)), ((), ())),
                          preferred_element_type=jnp.float32)
    kn2 = lax.dot_general(kb * kb, blockdiag, (((1,), (0,)), ((), ())),
                          preferred_element_type=jnp.float32)
    k2max = jnp.max(kn2, axis=0, keepdims=True)
    mpair = (qn2 + k2max) * 0.505

    zpad = jnp.zeros((T, d - 1), jnp.bfloat16)
    ves = [jnp.concatenate([vb[:, j * d:(j + 1) * d], ones_col, zpad], axis=1)
           for j in range(2)]
    units = [(qi, j) for qi in range(nq) for j in range(2)]
    sds, sps, mqs = {}, {}, {}
    for qi, j in units:
        lo = qi * tq
        qt = qb[lo:lo + tq, j * d:(j + 1) * d]
        kh = kb[:, j * d:(j + 1) * d]
        mqs[qi, j] = mpair[lo:lo + tq, j:j + 1]
        sds[qi, j] = lax.dot_general(qt, kh[lo:lo + tq],
                                     (((1,), (1,)), ((), ())),
                                     preferred_element_type=jnp.float32)
        if qi > 0:
            sps[qi, j] = lax.dot_general(qt, kh[:lo], (((1,), (1,)), ((), ())),
                                         preferred_element_type=jnp.float32)
    pds, pps = {}, {}
    for qi, j in units:
        pds[qi, j] = jnp.exp(sds[qi, j] - mqs[qi, j]).astype(jnp.bfloat16) * maskb
        if qi > 0:
            pps[qi, j] = jnp.exp(sps[qi, j] - mqs[qi, j]).astype(jnp.bfloat16)
    ys = [[], []]
    for qi, j in units:
        lo = qi * tq
        ya = lax.dot_general(pds[qi, j], ves[j][lo:lo + tq],
                             (((1,), (0,)), ((), ())),
                             preferred_element_type=jnp.float32)
        if qi > 0:
            ya = ya + lax.dot_general(pps[qi, j], ves[j][:lo],
                                      (((1,), (0,)), ((), ())),
                                      preferred_element_type=jnp.float32)
        y128 = ya * pl.reciprocal(ya[:, d:d + 1], approx=True)
        ys[j].append(y128.astype(jnp.bfloat16)[:, :d])
    outs = [jnp.concatenate(t, axis=0) for t in ys]
    y_ref[:, pl.ds(hp * 2 * d, 2 * d)] = jnp.concatenate(outs, axis=1)

    @pl.when(hp == pl.num_programs(1) - 1)
    def _project():
        o_ref[...] = jnp.dot(y_ref[...], wo_ref[...],
                             preferred_element_type=jnp.float32)


def kernel(x, w_attn, w_proj):
    B, T, C = x.shape
    n_head = 12
    D = C // n_head
    HP = n_head // 2
    scale = 1.0 / math.sqrt(D)
    tq = 256 if T % 256 == 0 else T

    scale_vec = jnp.concatenate(
        [jnp.full((2 * C, 1), scale ** 0.5, jnp.float32),
         jnp.ones((C, 1), jnp.float32)])
    w_packed = ((w_attn * scale_vec).astype(jnp.bfloat16).T
                .reshape(C, 3, HP, 2 * D)
                .transpose(0, 2, 1, 3)
                .reshape(C, 3 * C))
    w_out = w_proj.astype(jnp.bfloat16).T

    attn_cost = pl.CostEstimate(
        flops=(2 * B * T * C * 3 * C + 2 * 2 * B * n_head * T * T * D
               + 2 * B * T * C * C),
        transcendentals=B * n_head * T * T * 5 // 8,
        bytes_accessed=(B * T * C * 4) + (4 * C * C * 2) + (B * T * C * 4),
    )
    out = pl.pallas_call(
        functools.partial(_attn_pair_kernel, tq=tq, d=D),
        out_shape=jax.ShapeDtypeStruct((B, T, C), jnp.float32),
        grid=(B, HP),
        in_specs=[
            pl.BlockSpec((None, T, C), lambda b, hp: (b, 0, 0)),
            pl.BlockSpec((C, 6 * D), lambda b, hp: (0, hp)),
            pl.BlockSpec((C, C), lambda b, hp: (0, 0)),
        ],
        out_specs=pl.BlockSpec((None, T, C), lambda b, hp: (b, 0, 0)),
        scratch_shapes=[
            pltpu.VMEM((T, C), jnp.bfloat16),
            pltpu.VMEM((T, C), jnp.bfloat16),
        ],
        compiler_params=pltpu.CompilerParams(
            dimension_semantics=("parallel", "arbitrary"),
            vmem_limit_bytes=_VMEM_LIMIT,
        ),
        cost_estimate=attn_cost,
    )(x, w_packed, w_out)
    return out


# speedup vs baseline: 13.3767x; 1.0540x over previous
import functools
import math

import jax
import jax.numpy as jnp
from jax import lax
from jax.experimental import pallas as pl
from jax.experimental.pallas import tpu as pltpu

_VMEM_LIMIT = 48 * 1024 * 1024


def _attn_pair_kernel(x_ref, w_ref, wo_ref, o_ref, xb_ref, y_ref, *, tq, d):
    hp = pl.program_id(1)

    @pl.when(hp == 0)
    def _cast_x():
        xb_ref[...] = x_ref[...].astype(jnp.bfloat16)

    qkv = lax.dot_general(xb_ref[...], w_ref[...],
                          (((1,), (1,)), ((), ())),
                          preferred_element_type=jnp.float32
                          ).astype(jnp.bfloat16)
    T = qkv.shape[0]
    nq = T // tq
    row = lax.broadcasted_iota(jnp.int32, (tq, tq), 0)
    col = lax.broadcasted_iota(jnp.int32, (tq, tq), 1)
    maskb = jnp.where(col <= row, 1.0, 0.0).astype(jnp.bfloat16)
    ones_col = jnp.ones((T, 1), jnp.bfloat16)

    qb = qkv[:, 0:2 * d]
    kb = qkv[:, 2 * d:4 * d]
    vb = qkv[:, 4 * d:6 * d]
    seg = lax.broadcasted_iota(jnp.int32, (2 * d, 2), 0) // d
    hid = lax.broadcasted_iota(jnp.int32, (2 * d, 2), 1)
    blockdiag = jnp.where(seg == hid, 1.0, 0.0).astype(jnp.bfloat16)
    qn2 = lax.dot_general(qb * qb, blockdiag, (((1,), (0,)), ((), ())),
                          preferred_element_type=jnp.float32)
    kn2 = lax.dot_general(kb * kb, blockdiag, (((1,), (0,)), ((), ())),
                          preferred_element_type=jnp.float32)
    k2max = jnp.max(kn2, axis=0, keepdims=True)
    mpair = (qn2 + k2max) * 0.505

    zpad = jnp.zeros((T, d - 1), jnp.bfloat16)
    ves = [jnp.concatenate([vb[:, j * d:(j + 1) * d], ones_col, zpad], axis=1)
           for j in range(2)]
    units = [(qi, j) for qi in range(nq) for j in range(2)]
    sds, sps, mqs = {}, {}, {}
    for qi, j in units:
        lo = qi * tq
        qt = qb[lo:lo + tq, j * d:(j + 1) * d]
        kh = kb[:, j * d:(j + 1) * d]
        mqs[qi, j] = mpair[lo:lo + tq, j:j + 1]
        sds[qi, j] = lax.dot_general(qt, kh[lo:lo + tq],
                                     (((1,), (1,)), ((), ())),
                                     preferred_element_type=jnp.float32)
        if qi > 0:
            sps[qi, j] = lax.dot_general(qt, kh[:lo], (((1,), (1,)), ((), ())),
                                         preferred_element_type=jnp.float32)
    pds, pps = {}, {}
    for qi, j in units:
        pds[qi, j] = jnp.exp(sds[qi, j] - mqs[qi, j]).astype(jnp.bfloat16) * maskb
        if qi > 0:
            pps[qi, j] = jnp.exp(sps[qi, j] - mqs[qi, j]).astype(jnp.bfloat16)
    ys = [[], []]
    for qi, j in units:
        lo = qi * tq
        ya = lax.dot_general(pds[qi, j], ves[j][lo:lo + tq],
                             (((1,), (0,)), ((), ())),
                             preferred_element_type=jnp.float32)
        if qi > 0:
            ya = ya + lax.dot_general(pps[qi, j], ves[j][:lo],
                                      (((1,), (0,)), ((), ())),
                                      preferred_element_type=jnp.float32)
        y128 = ya * pl.reciprocal(ya[:, d:d + 1], approx=True)
        ys[j].append(y128.astype(jnp.bfloat16)[:, :d])
    outs = [jnp.concatenate(t, axis=0) for t in ys]
    y_ref[:, pl.ds(hp * 2 * d, 2 * d)] = jnp.concatenate(outs, axis=1)

    @pl.when(hp == pl.num_programs(1) - 1)
    def _project():
        o_ref[...] = lax.dot_general(y_ref[...], wo_ref[...],
                                     (((1,), (1,)), ((), ())),
                                     preferred_element_type=jnp.float32)


def kernel(x, w_attn, w_proj):
    B, T, C = x.shape
    n_head = 12
    D = C // n_head
    HP = n_head // 2
    scale = 1.0 / math.sqrt(D)
    tq = 256 if T % 256 == 0 else T

    scale_vec = jnp.concatenate(
        [jnp.full((2 * C, 1), scale ** 0.5, jnp.float32),
         jnp.ones((C, 1), jnp.float32)])
    w_packed = ((w_attn * scale_vec).astype(jnp.bfloat16)
                .reshape(3, HP, 2 * D, C)
                .transpose(1, 0, 2, 3)
                .reshape(3 * C, C))
    w_out = w_proj.astype(jnp.bfloat16)

    attn_cost = pl.CostEstimate(
        flops=(2 * B * T * C * 3 * C + 2 * 2 * B * n_head * T * T * D
               + 2 * B * T * C * C),
        transcendentals=B * n_head * T * T * 5 // 8,
        bytes_accessed=(B * T * C * 4) + (4 * C * C * 2) + (B * T * C * 4),
    )
    out = pl.pallas_call(
        functools.partial(_attn_pair_kernel, tq=tq, d=D),
        out_shape=jax.ShapeDtypeStruct((B, T, C), jnp.float32),
        grid=(B, HP),
        in_specs=[
            pl.BlockSpec((None, T, C), lambda b, hp: (b, 0, 0)),
            pl.BlockSpec((6 * D, C), lambda b, hp: (hp, 0)),
            pl.BlockSpec((C, C), lambda b, hp: (0, 0)),
        ],
        out_specs=pl.BlockSpec((None, T, C), lambda b, hp: (b, 0, 0)),
        scratch_shapes=[
            pltpu.VMEM((T, C), jnp.bfloat16),
            pltpu.VMEM((T, C), jnp.bfloat16),
        ],
        compiler_params=pltpu.CompilerParams(
            dimension_semantics=("parallel", "arbitrary"),
            vmem_limit_bytes=_VMEM_LIMIT,
        ),
        cost_estimate=attn_cost,
    )(x, w_packed, w_out)
    return out
